# baseline (speedup 1.0000x reference)
import functools

import jax
import jax.numpy as jnp
from jax import lax
from jax.experimental import pallas as pl
from jax.experimental.pallas import tpu as pltpu
from jax.experimental.pallas import tpu_sc as plsc

D = 128
B = 4096 * 50
CH = 128
NROWS = B // CH
NW = 32
RPW = NROWS // NW

_mesh = plsc.VectorSubcoreMesh(core_axis_name="c", subcore_axis_name="s")


NBUF = 4
MAIN = (RPW // NBUF) * NBUF


@functools.partial(
    pl.kernel,
    mesh=_mesh,
    out_type=jax.ShapeDtypeStruct((B, D), jnp.float32),
    scratch_types=[
        pltpu.VMEM((RPW, CH), jnp.int32),
    ]
    + [pltpu.VMEM((CH, D), jnp.float32) for _ in range(NBUF)]
    + [pltpu.SemaphoreType.DMA for _ in range(2 * NBUF)],
)
def _gather_kernel(idx_hbm, table_hbm, out_hbm, idx_v, *bufs_and_sems):
    bufs = bufs_and_sems[:NBUF]
    gsem = bufs_and_sems[NBUF : 2 * NBUF]
    wsem = bufs_and_sems[2 * NBUF :]
    wid = lax.axis_index("s") * 2 + lax.axis_index("c")
    row_base = wid * RPW
    pltpu.sync_copy(idx_hbm.at[wid], idx_v)

    def gather(j, k):
        return pltpu.make_async_copy(table_hbm.at[idx_v.at[j]], bufs[k], gsem[k])

    def writeback(j, k):
        return pltpu.make_async_copy(
            bufs[k], out_hbm.at[pl.ds((row_base + j) * CH, CH)], wsem[k]
        )

    for k in range(NBUF):
        gather(k, k).start()

    def body(i, carry):
        c0 = NBUF * i
        for k in range(NBUF):
            gather(c0 + k, k).wait()
            writeback(c0 + k, k).start()
        for k in range(NBUF):
            writeback(c0 + k, k).wait()

            @pl.when(c0 + NBUF + k < MAIN)
            def _():
                gather(c0 + NBUF + k, k).start()

        return carry

    lax.fori_loop(0, MAIN // NBUF, body, 0)

    for j in range(MAIN, RPW):
        k = j - MAIN
        gather(j, k).start()
        gather(j, k).wait()
        writeback(j, k).start()
        writeback(j, k).wait()


def kernel(input, weight):
    idx = input.reshape(NW, RPW, CH)
    out = _gather_kernel(idx, weight)
    return out.reshape(input.shape[0], input.shape[1], D)


# speedup vs baseline: 1.5585x; 1.5585x over previous
import functools

import jax
import jax.numpy as jnp
from jax import lax
from jax.experimental import pallas as pl
from jax.experimental.pallas import tpu as pltpu
from jax.experimental.pallas import tpu_sc as plsc

BATCH = 4096
HIST = 50
HPAD = 56
D = 128
NW = 32
BPW = BATCH // NW

_mesh = plsc.VectorSubcoreMesh(core_axis_name="c", subcore_axis_name="s")


@functools.partial(
    pl.kernel,
    mesh=_mesh,
    out_type=jax.ShapeDtypeStruct((BATCH, HIST, D), jnp.float32),
    scratch_types=[
        pltpu.VMEM((BPW, HPAD), jnp.int32),
        pltpu.VMEM((HIST, D), jnp.float32),
        pltpu.VMEM((HIST, D), jnp.float32),
        pltpu.SemaphoreType.DMA,
        pltpu.SemaphoreType.DMA,
    ],
)
def _gather_kernel(idx_hbm, table_hbm, out_hbm, idx_v, buf0, buf1, g0, g1):
    wid = lax.axis_index("s") * 2 + lax.axis_index("c")
    b0 = wid * BPW
    pltpu.sync_copy(idx_hbm.at[pl.ds(b0, BPW)], idx_v)

    def gather(m, buf, sem):
        return pltpu.make_async_copy(
            table_hbm.at[idx_v.at[m, pl.ds(0, HIST)]], buf, sem
        )

    def writeback(m, buf):
        pltpu.sync_copy(buf, out_hbm.at[b0 + m])

    gather(0, buf0, g0).start()

    def body(i, carry):
        m0 = 2 * i
        gather(m0 + 1, buf1, g1).start()
        gather(m0, buf0, g0).wait()
        writeback(m0, buf0)

        @pl.when(i < BPW // 2 - 1)
        def _():
            gather(m0 + 2, buf0, g0).start()

        gather(m0 + 1, buf1, g1).wait()
        writeback(m0 + 1, buf1)
        return carry

    lax.fori_loop(0, BPW // 2, body, 0)


def kernel(input, weight):
    idx = jnp.pad(input, ((0, 0), (0, HPAD - HIST)))
    return _gather_kernel(idx, weight)
